# Initial kernel scaffold; baseline (speedup 1.0000x reference)
import jax
import jax.numpy as jnp
from jax.experimental import pallas as pl


def kernel(z_query, key_proj_w, out_proj_w, memory):
    raise NotImplementedError("write your pallas kernel here")


# speedup vs baseline: 7.5402x; 7.5402x over previous
import functools

import jax
import jax.numpy as jnp
from jax.experimental import pallas as pl
from jax.experimental.pallas import tpu as pltpu

D = 512
H = 16
ROW_TILE = 2048


def _combine_kernel(kp_ref, mem_ref, wout_ref, e_ref, acc_ref):
    h = pl.program_id(0)

    @pl.when(h == 0)
    def _():
        acc_ref[...] = jnp.zeros_like(acc_ref)

    acc_ref[...] += jax.lax.dot_general(
        kp_ref[...], mem_ref[0],
        (((0,), (0,)), ((), ())),
        preferred_element_type=jnp.float32,
    )

    @pl.when(h == H - 1)
    def _():
        e_ref[...] = jax.lax.dot_general(
            acc_ref[...] * (1.0 / H), wout_ref[...],
            (((1,), (1,)), ((), ())),
            preferred_element_type=jnp.float32,
        )


def _apply_kernel(z_ref, e_ref, out_ref):
    out_ref[...] = jnp.dot(
        z_ref[...], e_ref[...], preferred_element_type=jnp.float32
    )


@jax.jit
def kernel(z_query, key_proj_w, out_proj_w, memory):
    e = pl.pallas_call(
        _combine_kernel,
        grid=(H,),
        in_specs=[
            pl.BlockSpec((D, D), lambda h: (h, 0)),
            pl.BlockSpec((1, D, D), lambda h: (h, 0, 0)),
            pl.BlockSpec((D, D), lambda h: (0, 0)),
        ],
        out_specs=pl.BlockSpec((D, D), lambda h: (0, 0)),
        out_shape=jax.ShapeDtypeStruct((D, D), jnp.float32),
        scratch_shapes=[pltpu.VMEM((D, D), jnp.float32)],
    )(key_proj_w, memory, out_proj_w)

    b = z_query.shape[0]
    out = pl.pallas_call(
        _apply_kernel,
        grid=(b // ROW_TILE,),
        in_specs=[
            pl.BlockSpec((ROW_TILE, D), lambda i: (i, 0)),
            pl.BlockSpec((D, D), lambda i: (0, 0)),
        ],
        out_specs=pl.BlockSpec((ROW_TILE, D), lambda i: (i, 0)),
        out_shape=jax.ShapeDtypeStruct((b, D), jnp.float32),
    )(z_query, e)
    return out


# speedup vs baseline: 8.4544x; 1.1212x over previous
import jax
import jax.numpy as jnp
from jax.experimental import pallas as pl
from jax.experimental.pallas import tpu as pltpu

D = 512
H = 16
K_TILE = 1024
N_K = (H * D) // K_TILE
ROW_TILE = 2048


def _combine_kernel(kp_ref, mem_ref, wout_ref, e_ref, acc_ref):
    i = pl.program_id(0)

    @pl.when(i == 0)
    def _():
        acc_ref[...] = jnp.zeros_like(acc_ref)

    acc_ref[...] += jax.lax.dot_general(
        kp_ref[...], mem_ref[...],
        (((0,), (0,)), ((), ())),
        preferred_element_type=jnp.float32,
    )

    @pl.when(i == N_K - 1)
    def _():
        e_ref[...] = jax.lax.dot_general(
            acc_ref[...] * (1.0 / H), wout_ref[...],
            (((1,), (1,)), ((), ())),
            preferred_element_type=jnp.float32,
        )


def _apply_kernel(z_ref, e_ref, out_ref):
    out_ref[...] = jnp.dot(
        z_ref[...], e_ref[...], preferred_element_type=jnp.float32
    )


@jax.jit
def kernel(z_query, key_proj_w, out_proj_w, memory):
    mem_2d = memory.reshape(H * D, D)
    e = pl.pallas_call(
        _combine_kernel,
        grid=(N_K,),
        in_specs=[
            pl.BlockSpec((K_TILE, D), lambda i: (i, 0)),
            pl.BlockSpec((K_TILE, D), lambda i: (i, 0)),
            pl.BlockSpec((D, D), lambda i: (0, 0)),
        ],
        out_specs=pl.BlockSpec((D, D), lambda i: (0, 0)),
        out_shape=jax.ShapeDtypeStruct((D, D), jnp.float32),
        scratch_shapes=[pltpu.VMEM((D, D), jnp.float32)],
    )(key_proj_w, mem_2d, out_proj_w)

    b = z_query.shape[0]
    out = pl.pallas_call(
        _apply_kernel,
        grid=(b // ROW_TILE,),
        in_specs=[
            pl.BlockSpec((ROW_TILE, D), lambda i: (i, 0)),
            pl.BlockSpec((D, D), lambda i: (0, 0)),
        ],
        out_specs=pl.BlockSpec((ROW_TILE, D), lambda i: (i, 0)),
        out_shape=jax.ShapeDtypeStruct((b, D), jnp.float32),
    )(z_query, e)
    return out


# speedup vs baseline: 8.8792x; 1.0503x over previous
import jax
import jax.numpy as jnp
from jax.experimental import pallas as pl
from jax.experimental.pallas import tpu as pltpu

D = 512
H = 16
K_TILE = 1024
N_K = (H * D) // K_TILE
ROW_TILE = 2048


def _fused_kernel(kp_ref, mem_ref, wout_ref, z_ref, out_ref, acc_ref, e_ref):
    i = pl.program_id(0)

    @pl.when(i == 0)
    def _():
        acc_ref[...] = jnp.zeros_like(acc_ref)

    @pl.when(i < N_K)
    def _():
        acc_ref[...] += jax.lax.dot_general(
            kp_ref[...], mem_ref[...],
            (((0,), (0,)), ((), ())),
            preferred_element_type=jnp.float32,
        )

    @pl.when(i == N_K - 1)
    def _():
        e_ref[...] = jax.lax.dot_general(
            acc_ref[...] * (1.0 / H), wout_ref[...],
            (((1,), (1,)), ((), ())),
            preferred_element_type=jnp.float32,
        )

    @pl.when(i >= N_K)
    def _():
        out_ref[...] = jnp.dot(
            z_ref[...], e_ref[...], preferred_element_type=jnp.float32
        )


@jax.jit
def kernel(z_query, key_proj_w, out_proj_w, memory):
    mem_2d = memory.reshape(H * D, D)
    b = z_query.shape[0]
    n_b = b // ROW_TILE
    out = pl.pallas_call(
        _fused_kernel,
        grid=(N_K + n_b,),
        in_specs=[
            pl.BlockSpec((K_TILE, D), lambda i: (jnp.minimum(i, N_K - 1), 0)),
            pl.BlockSpec((K_TILE, D), lambda i: (jnp.minimum(i, N_K - 1), 0)),
            pl.BlockSpec((D, D), lambda i: (0, 0)),
            pl.BlockSpec((ROW_TILE, D), lambda i: (jnp.maximum(i - N_K, 0), 0)),
        ],
        out_specs=pl.BlockSpec((ROW_TILE, D), lambda i: (jnp.maximum(i - N_K, 0), 0)),
        out_shape=jax.ShapeDtypeStruct((b, D), jnp.float32),
        scratch_shapes=[
            pltpu.VMEM((D, D), jnp.float32),
            pltpu.VMEM((D, D), jnp.float32),
        ],
    )(key_proj_w, mem_2d, out_proj_w, z_query)
    return out
